# baseline (speedup 1.0000x reference)
import functools
import math

import jax
import jax.numpy as jnp
from jax import lax
from jax.experimental import pallas as pl
from jax.experimental.pallas import tpu as pltpu
from jax.experimental.pallas import tpu_sc as plsc

_N = 10000
_E = 320000
_H = 8
_D = 32
_HD = _H * _D
_HF = _HD // 2

_NC = 2
_NS = 16
_NW = _NC * _NS

_P1_EPW = _E // _NW
_EB1 = 16
_NB1 = _P1_EPW // _EB1
_PAIRS1 = (_NB1 - 1) // 2

_P2_EPT = _E // _NS
_EB2 = 80
_NB2 = _P2_EPT // _EB2
_PAIRS2 = _NB2 // 2

_ROWS_PT = _N // _NS
_ZROWS = 25

_ROW_BLK = 1000

_SC_PARAMS = pltpu.CompilerParams(
    use_tc_tiling_on_sc=False, needs_layout_passes=False)


def _tables_body(x_ref, w1_ref, w2_ref, b1_ref, b2_ref,
                 src_ref, dst_ref, h1a_ref, h1b_ref):
    x = x_ref[...]
    dn = (((1,), (1,)), ((), ()))
    h1 = lax.dot_general(x, w1_ref[...], dn,
                         preferred_element_type=jnp.float32) + b1_ref[...]
    h2 = lax.dot_general(x, w2_ref[...], dn,
                         preferred_element_type=jnp.float32) + b2_ref[...]
    hp = h1 * h2
    src_ref[...] = jnp.concatenate([h1, hp], axis=1)
    dst_ref[...] = jnp.concatenate([h2, hp], axis=1)
    h1a_ref[...] = h1[:, :_HF]
    h1b_ref[...] = h1[:, _HF:]


def _tables(x, w1, w2, b1, b2):
    n, k = x.shape
    r = _ROW_BLK
    return pl.pallas_call(
        _tables_body,
        grid=(n // r,),
        in_specs=[
            pl.BlockSpec((r, k), lambda i: (i, 0)),
            pl.BlockSpec((_HD, k), lambda i: (0, 0)),
            pl.BlockSpec((_HD, k), lambda i: (0, 0)),
            pl.BlockSpec((1, _HD), lambda i: (0, 0)),
            pl.BlockSpec((1, _HD), lambda i: (0, 0)),
        ],
        out_specs=[
            pl.BlockSpec((r, 2 * _HD), lambda i: (i, 0)),
            pl.BlockSpec((r, 2 * _HD), lambda i: (i, 0)),
            pl.BlockSpec((r, _HF), lambda i: (i, 0)),
            pl.BlockSpec((r, _HF), lambda i: (i, 0)),
        ],
        out_shape=[
            jax.ShapeDtypeStruct((n, 2 * _HD), jnp.float32),
            jax.ShapeDtypeStruct((n, 2 * _HD), jnp.float32),
            jax.ShapeDtypeStruct((n, _HF), jnp.float32),
            jax.ShapeDtypeStruct((n, _HF), jnp.float32),
        ],
    )(x, w1, w2, b1.reshape(1, -1), b2.reshape(1, -1))


def _ln_elu_body(has_res, ha_ref, hb_ref, g_ref, be_ref, *rest):
    if has_res:
        res_ref, o_ref = rest
    else:
        (o_ref,) = rest
    h = jnp.concatenate([ha_ref[0], hb_ref[0]], axis=1)
    m = jnp.mean(h, axis=1, keepdims=True)
    xm = h - m
    v = jnp.mean(xm * xm, axis=1, keepdims=True)
    y = xm * lax.rsqrt(v + 1e-5) * g_ref[...] + be_ref[...]
    y = jnp.where(y > 0, y, jnp.exp(y) - 1.0)
    if has_res:
        y = y + res_ref[...]
    o_ref[...] = y


def _ln_elu(gat2, g, b, res):
    r = _ROW_BLK
    has_res = res is not None
    in_specs = [
        pl.BlockSpec((1, r, _HF), lambda i: (0, i, 0)),
        pl.BlockSpec((1, r, _HF), lambda i: (1, i, 0)),
        pl.BlockSpec((1, _HD), lambda i: (0, 0)),
        pl.BlockSpec((1, _HD), lambda i: (0, 0)),
    ]
    args = [gat2, gat2, g.reshape(1, -1), b.reshape(1, -1)]
    if has_res:
        in_specs.append(pl.BlockSpec((r, _HD), lambda i: (i, 0)))
        args.append(res)
    return pl.pallas_call(
        functools.partial(_ln_elu_body, has_res),
        grid=(_N // r,),
        in_specs=in_specs,
        out_specs=pl.BlockSpec((r, _HD), lambda i: (i, 0)),
        out_shape=jax.ShapeDtypeStruct((_N, _HD), jnp.float32),
    )(*args)


def _final_body(h_ref, w_ref, b_ref, o_ref):
    dn = (((1,), (1,)), ((), ()))
    o_ref[...] = lax.dot_general(h_ref[...], w_ref[...], dn,
                                 preferred_element_type=jnp.float32) + b_ref[...]


def _final(h, w_out, b_out):
    r = _ROW_BLK
    d_out = w_out.shape[0]
    return pl.pallas_call(
        _final_body,
        grid=(_N // r,),
        in_specs=[
            pl.BlockSpec((r, _HD), lambda i: (i, 0)),
            pl.BlockSpec((d_out, _HD), lambda i: (0, 0)),
            pl.BlockSpec((1, d_out), lambda i: (0, 0)),
        ],
        out_specs=pl.BlockSpec((r, d_out), lambda i: (i, 0)),
        out_shape=jax.ShapeDtypeStruct((_N, d_out), jnp.float32),
    )(h, w_out, b_out.reshape(1, -1))


def _attn_sc(src_tab, dst_tab, esrc, edst, a_scaled):
    mesh = plsc.VectorSubcoreMesh(core_axis_name="c", subcore_axis_name="s")

    @functools.partial(
        pl.kernel,
        out_type=jax.ShapeDtypeStruct((_E * _H,), jnp.float32),
        mesh=mesh,
        compiler_params=_SC_PARAMS,
        scratch_types=[
            pltpu.VMEM((_P1_EPW,), jnp.int32),
            pltpu.VMEM((_P1_EPW,), jnp.int32),
            pltpu.VMEM((4 * _EB1, _HF), jnp.float32),
            pltpu.VMEM((4 * _EB1, _HF), jnp.float32),
            pltpu.VMEM((4 * _EB1, _HF), jnp.float32),
            pltpu.VMEM((4 * _EB1, _HF), jnp.float32),
            pltpu.VMEM((4 * _EB1,), jnp.int32),
            pltpu.VMEM((4 * _EB1,), jnp.int32),
            pltpu.VMEM((4 * _EB1,), jnp.int32),
            pltpu.VMEM((4 * _EB1,), jnp.int32),
            pltpu.VMEM((_EB1 * _H,), jnp.float32),
            pltpu.VMEM((_EB1 * _H,), jnp.float32),
            pltpu.VMEM((_HD,), jnp.float32),
            pltpu.SemaphoreType.DMA,
            pltpu.SemaphoreType.DMA,
            pltpu.SemaphoreType.DMA,
            pltpu.SemaphoreType.DMA,
        ],
    )
    def k(src_hbm, dst_hbm, esrc_hbm, edst_hbm, a_hbm, attn_hbm,
          esrc_v, edst_v, sr_a, sr_b, dr_a, dr_b, ixs_a, ixs_b, ixd_a, ixd_b,
          at_a, at_b, a_v, gs_a, gs_b, ws_a, ws_b):
        wid = lax.axis_index("s") * _NC + lax.axis_index("c")
        ebase = wid * _P1_EPW
        pltpu.sync_copy(esrc_hbm.at[pl.ds(ebase, _P1_EPW)], esrc_v)
        pltpu.sync_copy(edst_hbm.at[pl.ds(ebase, _P1_EPW)], edst_v)
        pltpu.sync_copy(a_hbm, a_v)
        lanes = lax.iota(jnp.int32, 16)
        lanes_h = lanes * _H
        lanes4 = lanes * 4
        zero16 = jnp.zeros((16,), jnp.float32)

        def issue(b, sr, dr, ixs, ixd, gs):
            off = b * _EB1
            es4 = esrc_v[pl.ds(off, _EB1)] * 4
            ed4 = edst_v[pl.ds(off, _EB1)] * 4
            for kk in range(4):
                plsc.store_scatter(ixs, [lanes4 + kk], es4 + kk)
                plsc.store_scatter(ixd, [lanes4 + kk], ed4 + kk)
            pltpu.async_copy(src_hbm.at[ixs], sr, gs)
            pltpu.async_copy(dst_hbm.at[ixd], dr, gs)

        def wait_gather(sr, dr, gs):
            pltpu.make_async_copy(src_hbm.at[pl.ds(0, 4 * _EB1)], sr, gs).wait()
            pltpu.make_async_copy(dst_hbm.at[pl.ds(0, 4 * _EB1)], dr, gs).wait()

        def drain_at(at, ws):
            pltpu.make_async_copy(
                at, attn_hbm.at[pl.ds(0, _EB1 * _H)], ws).wait()

        def compute(b, sr, dr, at, ws, wait_pred):
            accs = tuple(zero16 for _ in range(_H))

            def dbody(d, accs):
                out = []
                for h in range(_H):
                    colv = jnp.full((16,), d + (h % 4) * _D, jnp.int32)
                    colva = jnp.full((16,), d + h * _D, jnp.int32)
                    h1s = plsc.load_gather(sr, [lanes4 + (h // 4), colv])
                    hps = plsc.load_gather(sr, [lanes4 + (2 + h // 4), colv])
                    h2d = plsc.load_gather(dr, [lanes4 + (h // 4), colv])
                    hpd = plsc.load_gather(dr, [lanes4 + (2 + h // 4), colv])
                    z = h1s + h2d + hps * hpd
                    ez = jnp.where(z > 0, z, jnp.exp(z) - 1.0)
                    av = plsc.load_gather(a_v, [colva])
                    out.append(accs[h] + av * ez)
                return tuple(out)

            accs = lax.fori_loop(0, _D, dbody, accs)
            m = accs[0]
            for h in range(1, _H):
                m = jnp.maximum(m, accs[h])
            es = [jnp.exp(v - m) for v in accs]
            tot = es[0]
            for h in range(1, _H):
                tot = tot + es[h]
            r = 1.0 / tot

            @pl.when(wait_pred)
            def _():
                drain_at(at, ws)

            for h in range(_H):
                plsc.store_scatter(at, [lanes_h + h], es[h] * r)
            pltpu.async_copy(
                at, attn_hbm.at[pl.ds((ebase + b * _EB1) * _H, _EB1 * _H)], ws)

        issue(0, sr_a, dr_a, ixs_a, ixd_a, gs_a)

        def pair(g, carry):
            b0 = 2 * g
            issue(b0 + 1, sr_b, dr_b, ixs_b, ixd_b, gs_b)
            wait_gather(sr_a, dr_a, gs_a)
            compute(b0, sr_a, dr_a, at_a, ws_a, g > 0)
            issue(b0 + 2, sr_a, dr_a, ixs_a, ixd_a, gs_a)
            wait_gather(sr_b, dr_b, gs_b)
            compute(b0 + 1, sr_b, dr_b, at_b, ws_b, g > 0)
            return carry

        lax.fori_loop(0, _PAIRS1, pair, 0)
        wait_gather(sr_a, dr_a, gs_a)
        compute(_NB1 - 1, sr_a, dr_a, at_a, ws_a, _PAIRS1 > 0)
        drain_at(at_a, ws_a)
        drain_at(at_b, ws_b)

    return k(src_tab, dst_tab, esrc, edst, a_scaled)


def _agg_sc(h1cat, esrc, edst, attn):
    mesh = plsc.VectorSubcoreMesh(core_axis_name="c", subcore_axis_name="s")

    @functools.partial(
        pl.kernel,
        out_type=jax.ShapeDtypeStruct((_NC, _N, _HF), jnp.float32),
        mesh=mesh,
        compiler_params=_SC_PARAMS,
        scratch_types=[
            pltpu.VMEM((_EB2, _HF), jnp.float32),
            pltpu.VMEM((_EB2, _HF), jnp.float32),
            pltpu.VMEM((_EB2 * _H,), jnp.float32),
            pltpu.VMEM((_EB2 * _H,), jnp.float32),
            pltpu.VMEM((_EB2, _HF), jnp.float32),
            pltpu.VMEM((_EB2, _HF), jnp.float32),
            pltpu.VMEM((_EB2,), jnp.int32),
            pltpu.VMEM((_EB2,), jnp.int32),
            pltpu.VMEM((_EB2,), jnp.int32),
            pltpu.VMEM((_EB2,), jnp.int32),
            pltpu.VMEM((_EB2,), jnp.int32),
            pltpu.VMEM((_EB2,), jnp.int32),
            pltpu.VMEM((_ZROWS, _HF), jnp.float32),
            pltpu.VMEM_SHARED((_N, _HF), jnp.float32),
            pltpu.SemaphoreType.DMA,
            pltpu.SemaphoreType.DMA,
            pltpu.SemaphoreType.DMA,
            pltpu.SemaphoreType.DMA,
            pltpu.SemaphoreType.DMA,
            pltpu.SemaphoreType.DMA,
        ],
    )
    def k(h1_hbm, esrc_hbm, edst_hbm, attn_hbm, out_hbm,
          rows_a, rows_b, at_a, at_b, msg_a, msg_b, sidx_a, sidx_b,
          didxf_a, didxf_b, didxu_a, didxu_b, zero_v, acc_sh,
          gs_a, gs_b, ss_a, ss_b, is_a, is_b):
        c = lax.axis_index("c")
        s = lax.axis_index("s")
        zvec = jnp.zeros((16,), jnp.float32)

        def zrow(i, carry):
            for kk in range(_HF // 16):
                zero_v[i, pl.ds(kk * 16, 16)] = zvec
            return carry

        lax.fori_loop(0, _ZROWS, zrow, 0)
        for j in range(_ROWS_PT // _ZROWS):
            pltpu.sync_copy(
                zero_v, acc_sh.at[pl.ds(s * _ROWS_PT + j * _ZROWS, _ZROWS)])
        plsc.subcore_barrier()

        ebase = s * _P2_EPT
        cn = c * _N
        hbase = c * (_H // 2)

        def idx_issue(b, sidx, didxf, isem):
            off = ebase + b * _EB2
            pltpu.async_copy(esrc_hbm.at[pl.ds(off, _EB2)], sidx, isem)
            pltpu.async_copy(edst_hbm.at[pl.ds(off, _EB2)], didxf, isem)

        def wait_idx(sidx, didxf, isem):
            pltpu.make_async_copy(
                esrc_hbm.at[pl.ds(0, _EB2)], sidx, isem).wait()
            pltpu.make_async_copy(
                edst_hbm.at[pl.ds(0, _EB2)], didxf, isem).wait()

        def gather_issue(b, sidx, rows, at, gs):
            for kk in range(_EB2 // 16):
                sidx[pl.ds(kk * 16, 16)] = sidx[pl.ds(kk * 16, 16)] + cn
            pltpu.async_copy(h1_hbm.at[sidx], rows, gs)
            pltpu.async_copy(
                attn_hbm.at[pl.ds((ebase + b * _EB2) * _H, _EB2 * _H)], at, gs)

        def wait_gather(rows, at, gs):
            pltpu.make_async_copy(h1_hbm.at[pl.ds(0, _EB2)], rows, gs).wait()
            pltpu.make_async_copy(
                attn_hbm.at[pl.ds(0, _EB2 * _H)], at, gs).wait()

        def wait_scatter(msg, didxu, ss):
            pltpu.make_async_copy(msg, acc_sh.at[didxu], ss).wait()

        def compute(rows, at, msg, didxf, didxu, ss):
            for kk in range(_EB2 // 16):
                didxu[pl.ds(kk * 16, 16)] = didxf[pl.ds(kk * 16, 16)]

            def ebody(i, carry):
                for k4 in range(4):
                    e = i * 4 + k4
                    e8 = e * _H
                    for hh in range(_H // 2):
                        aidx = jnp.full((16,), e8 + hbase + hh, jnp.int32)
                        av = plsc.load_gather(at, [aidx])
                        for q in range(2):
                            vv = hh * 2 + q
                            msg[e, pl.ds(vv * 16, 16)] = (
                                rows[e, pl.ds(vv * 16, 16)] * av)
                return carry

            lax.fori_loop(0, _EB2 // 4, ebody, 0)
            pltpu.async_copy(msg, acc_sh.at[didxu], ss, add=True)

        idx_issue(0, sidx_a, didxf_a, is_a)
        idx_issue(1, sidx_b, didxf_b, is_b)
        wait_idx(sidx_a, didxf_a, is_a)
        gather_issue(0, sidx_a, rows_a, at_a, gs_a)

        def pair(g, carry):
            b0 = 2 * g
            wait_idx(sidx_b, didxf_b, is_b)
            gather_issue(b0 + 1, sidx_b, rows_b, at_b, gs_b)

            @pl.when(g > 0)
            def _():
                wait_scatter(msg_a, didxu_a, ss_a)

            wait_gather(rows_a, at_a, gs_a)
            compute(rows_a, at_a, msg_a, didxf_a, didxu_a, ss_a)
            idx_issue(b0 + 2, sidx_a, didxf_a, is_a)
            wait_idx(sidx_a, didxf_a, is_a)
            gather_issue(b0 + 2, sidx_a, rows_a, at_a, gs_a)

            @pl.when(g > 0)
            def _():
                wait_scatter(msg_b, didxu_b, ss_b)

            wait_gather(rows_b, at_b, gs_b)
            compute(rows_b, at_b, msg_b, didxf_b, didxu_b, ss_b)
            idx_issue(b0 + 3, sidx_b, didxf_b, is_b)
            return carry

        lax.fori_loop(0, _PAIRS2 - 1, pair, 0)
        wait_idx(sidx_b, didxf_b, is_b)
        gather_issue(_NB2 - 1, sidx_b, rows_b, at_b, gs_b)
        wait_scatter(msg_a, didxu_a, ss_a)
        wait_gather(rows_a, at_a, gs_a)
        compute(rows_a, at_a, msg_a, didxf_a, didxu_a, ss_a)
        wait_scatter(msg_b, didxu_b, ss_b)
        wait_gather(rows_b, at_b, gs_b)
        compute(rows_b, at_b, msg_b, didxf_b, didxu_b, ss_b)
        wait_scatter(msg_a, didxu_a, ss_a)
        wait_scatter(msg_b, didxu_b, ss_b)
        plsc.subcore_barrier()
        pltpu.sync_copy(acc_sh.at[pl.ds(s * _ROWS_PT, _ROWS_PT)],
                        out_hbm.at[c, pl.ds(s * _ROWS_PT, _ROWS_PT)])

    return k(h1cat, esrc, edst, attn)


def _gat_layer(x, edge_index, w1, b1, w2, b2, a):
    esrc = edge_index[0]
    edst = edge_index[1]
    src_tab, dst_tab, h1a, h1b = _tables(x, w1, w2, b1, b2)
    h1cat = jnp.concatenate([h1a, h1b], axis=0)
    a_scaled = (a / math.sqrt(_D)).reshape(-1).astype(jnp.float32)
    attn = _attn_sc(src_tab.reshape(4 * _N, _HF), dst_tab.reshape(4 * _N, _HF),
                    esrc, edst, a_scaled)
    return _agg_sc(h1cat, esrc, edst, attn)


def kernel(x, edge_index, W1_0, b1_0, W2_0, b2_0, W3_0, b3_0, a_0, ln_g_0,
           ln_b_0, W1_1, b1_1, W2_1, b2_1, W3_1, b3_1, a_1, ln_g_1, ln_b_1,
           W_out, b_out):
    gat0 = _gat_layer(x, edge_index, W1_0, b1_0, W2_0, b2_0, a_0)
    h = _ln_elu(gat0, ln_g_0, ln_b_0, None)
    gat1 = _gat_layer(h, edge_index, W1_1, b1_1, W2_1, b2_1, a_1)
    h2 = _ln_elu(gat1, ln_g_1, ln_b_1, h)
    return _final(h2, W_out, b_out)


# speedup vs baseline: 1.6504x; 1.6504x over previous
import functools
import math

import jax
import jax.numpy as jnp
from jax import lax
from jax.experimental import pallas as pl
from jax.experimental.pallas import tpu as pltpu
from jax.experimental.pallas import tpu_sc as plsc

_N = 10000
_E = 320000
_H = 8
_D = 32
_HD = _H * _D
_HF = _HD // 2

_NC = 2
_NS = 16
_NW = _NC * _NS

_P1_EPW = _E // _NW
_EB1 = 16
_NB1 = _P1_EPW // _EB1
_PAIRS1 = (_NB1 - 1) // 2

_P2_EPT = _E // _NS
_EB2 = 80
_NB2 = _P2_EPT // _EB2
_PAIRS2 = _NB2 // 2

_ROWS_PT = _N // _NS
_ZROWS = 25

_ROW_BLK = 1000

_SC_PARAMS = pltpu.CompilerParams(
    use_tc_tiling_on_sc=False, needs_layout_passes=False)


def _tables_body(x_ref, w1_ref, w2_ref, b1_ref, b2_ref,
                 src_ref, dst_ref, h1a_ref, h1b_ref):
    x = x_ref[...]
    dn = (((1,), (1,)), ((), ()))
    h1 = lax.dot_general(x, w1_ref[...], dn,
                         preferred_element_type=jnp.float32) + b1_ref[...]
    h2 = lax.dot_general(x, w2_ref[...], dn,
                         preferred_element_type=jnp.float32) + b2_ref[...]
    hp = h1 * h2
    src_ref[...] = jnp.concatenate([h1, hp], axis=1)
    dst_ref[...] = jnp.concatenate([h2, hp], axis=1)
    h1a_ref[...] = h1[:, :_HF]
    h1b_ref[...] = h1[:, _HF:]


def _tables(x, w1, w2, b1, b2):
    n, k = x.shape
    r = _ROW_BLK
    return pl.pallas_call(
        _tables_body,
        grid=(n // r,),
        in_specs=[
            pl.BlockSpec((r, k), lambda i: (i, 0)),
            pl.BlockSpec((_HD, k), lambda i: (0, 0)),
            pl.BlockSpec((_HD, k), lambda i: (0, 0)),
            pl.BlockSpec((1, _HD), lambda i: (0, 0)),
            pl.BlockSpec((1, _HD), lambda i: (0, 0)),
        ],
        out_specs=[
            pl.BlockSpec((r, 2 * _HD), lambda i: (i, 0)),
            pl.BlockSpec((r, 2 * _HD), lambda i: (i, 0)),
            pl.BlockSpec((r, _HF), lambda i: (i, 0)),
            pl.BlockSpec((r, _HF), lambda i: (i, 0)),
        ],
        out_shape=[
            jax.ShapeDtypeStruct((n, 2 * _HD), jnp.float32),
            jax.ShapeDtypeStruct((n, 2 * _HD), jnp.float32),
            jax.ShapeDtypeStruct((n, _HF), jnp.float32),
            jax.ShapeDtypeStruct((n, _HF), jnp.float32),
        ],
    )(x, w1, w2, b1.reshape(1, -1), b2.reshape(1, -1))


def _ln_elu_body(has_res, ha_ref, hb_ref, g_ref, be_ref, *rest):
    if has_res:
        res_ref, o_ref = rest
    else:
        (o_ref,) = rest
    h = jnp.concatenate([ha_ref[0], hb_ref[0]], axis=1)
    m = jnp.mean(h, axis=1, keepdims=True)
    xm = h - m
    v = jnp.mean(xm * xm, axis=1, keepdims=True)
    y = xm * lax.rsqrt(v + 1e-5) * g_ref[...] + be_ref[...]
    y = jnp.where(y > 0, y, jnp.exp(y) - 1.0)
    if has_res:
        y = y + res_ref[...]
    o_ref[...] = y


def _ln_elu(gat2, g, b, res):
    r = _ROW_BLK
    has_res = res is not None
    in_specs = [
        pl.BlockSpec((1, r, _HF), lambda i: (0, i, 0)),
        pl.BlockSpec((1, r, _HF), lambda i: (1, i, 0)),
        pl.BlockSpec((1, _HD), lambda i: (0, 0)),
        pl.BlockSpec((1, _HD), lambda i: (0, 0)),
    ]
    args = [gat2, gat2, g.reshape(1, -1), b.reshape(1, -1)]
    if has_res:
        in_specs.append(pl.BlockSpec((r, _HD), lambda i: (i, 0)))
        args.append(res)
    return pl.pallas_call(
        functools.partial(_ln_elu_body, has_res),
        grid=(_N // r,),
        in_specs=in_specs,
        out_specs=pl.BlockSpec((r, _HD), lambda i: (i, 0)),
        out_shape=jax.ShapeDtypeStruct((_N, _HD), jnp.float32),
    )(*args)


def _final_body(h_ref, w_ref, b_ref, o_ref):
    dn = (((1,), (1,)), ((), ()))
    o_ref[...] = lax.dot_general(h_ref[...], w_ref[...], dn,
                                 preferred_element_type=jnp.float32) + b_ref[...]


def _final(h, w_out, b_out):
    r = _ROW_BLK
    d_out = w_out.shape[0]
    return pl.pallas_call(
        _final_body,
        grid=(_N // r,),
        in_specs=[
            pl.BlockSpec((r, _HD), lambda i: (i, 0)),
            pl.BlockSpec((d_out, _HD), lambda i: (0, 0)),
            pl.BlockSpec((1, d_out), lambda i: (0, 0)),
        ],
        out_specs=pl.BlockSpec((r, d_out), lambda i: (i, 0)),
        out_shape=jax.ShapeDtypeStruct((_N, d_out), jnp.float32),
    )(h, w_out, b_out.reshape(1, -1))


def _attn_sc(src_tab, dst_tab, esrc, edst, a_scaled):
    mesh = plsc.VectorSubcoreMesh(core_axis_name="c", subcore_axis_name="s")

    @functools.partial(
        pl.kernel,
        out_type=jax.ShapeDtypeStruct((_E * _H,), jnp.float32),
        mesh=mesh,
        compiler_params=_SC_PARAMS,
        scratch_types=[
            pltpu.VMEM((_P1_EPW,), jnp.int32),
            pltpu.VMEM((_P1_EPW,), jnp.int32),
            pltpu.VMEM((_EB1, _HD), jnp.int32),
            pltpu.VMEM((_EB1, _HD), jnp.int32),
            pltpu.VMEM((_EB1, _HD), jnp.int32),
            pltpu.VMEM((_EB1, _HD), jnp.int32),
            pltpu.VMEM((_EB1 * _H,), jnp.float32),
            pltpu.VMEM((_EB1 * _H,), jnp.float32),
            pltpu.VMEM((_HD,), jnp.float32),
            pltpu.SemaphoreType.DMA,
            pltpu.SemaphoreType.DMA,
            pltpu.SemaphoreType.DMA,
            pltpu.SemaphoreType.DMA,
        ],
    )
    def k(src_hbm, dst_hbm, esrc_hbm, edst_hbm, a_hbm, attn_hbm,
          esrc_v, edst_v, sr_a, sr_b, dr_a, dr_b,
          at_a, at_b, a_v, gs_a, gs_b, ws_a, ws_b):
        wid = lax.axis_index("s") * _NC + lax.axis_index("c")
        ebase = wid * _P1_EPW
        pltpu.sync_copy(esrc_hbm.at[pl.ds(ebase, _P1_EPW)], esrc_v)
        pltpu.sync_copy(edst_hbm.at[pl.ds(ebase, _P1_EPW)], edst_v)
        pltpu.sync_copy(a_hbm, a_v)
        lanes = lax.iota(jnp.int32, 16)
        lanes_h = lanes * _H
        zero16 = jnp.zeros((16,), jnp.float32)

        def issue(b, sr, dr, gs):
            off = b * _EB1
            pltpu.async_copy(src_hbm.at[esrc_v.at[pl.ds(off, _EB1)]], sr, gs)
            pltpu.async_copy(dst_hbm.at[edst_v.at[pl.ds(off, _EB1)]], dr, gs)

        def wait_gather(sr, dr, gs):
            pltpu.make_async_copy(src_hbm.at[pl.ds(0, _EB1)], sr, gs).wait()
            pltpu.make_async_copy(dst_hbm.at[pl.ds(0, _EB1)], dr, gs).wait()

        def drain_at(at, ws):
            pltpu.make_async_copy(
                at, attn_hbm.at[pl.ds(0, _EB1 * _H)], ws).wait()

        def compute(b, sr, dr, at, ws, wait_pred):
            accs = tuple(zero16 for _ in range(_H))

            def unp(w):
                return plsc.unpack(plsc.bitcast(w, jnp.bfloat16),
                                   format=plsc.PackFormat.INTERLEAVED,
                                   preferred_element_type=jnp.float32)

            def dbody(d2, accs):
                out = []
                for h in range(_H):
                    colv = jnp.full((16,), d2 + h * (_D // 2), jnp.int32)
                    colv2 = colv + _HD // 2
                    h1e, h1o = unp(plsc.load_gather(sr, [lanes, colv]))
                    hpe, hpo = unp(plsc.load_gather(sr, [lanes, colv2]))
                    h2e, h2o = unp(plsc.load_gather(dr, [lanes, colv]))
                    hqe, hqo = unp(plsc.load_gather(dr, [lanes, colv2]))
                    ze = h1e + h2e + hpe * hqe
                    zo = h1o + h2o + hpo * hqo
                    eze = jnp.where(ze > 0, ze, jnp.exp(ze) - 1.0)
                    ezo = jnp.where(zo > 0, zo, jnp.exp(zo) - 1.0)
                    colva = jnp.full((16,), 2 * d2 + h * _D, jnp.int32)
                    ave = plsc.load_gather(a_v, [colva])
                    avo = plsc.load_gather(a_v, [colva + 1])
                    out.append(accs[h] + (ave * eze + avo * ezo))
                return tuple(out)

            accs = lax.fori_loop(0, _D // 2, dbody, accs)
            m = accs[0]
            for h in range(1, _H):
                m = jnp.maximum(m, accs[h])
            es = [jnp.exp(v - m) for v in accs]
            tot = es[0]
            for h in range(1, _H):
                tot = tot + es[h]
            r = 1.0 / tot

            @pl.when(wait_pred)
            def _():
                drain_at(at, ws)

            for h in range(_H):
                plsc.store_scatter(at, [lanes_h + h], es[h] * r)
            pltpu.async_copy(
                at, attn_hbm.at[pl.ds((ebase + b * _EB1) * _H, _EB1 * _H)], ws)

        issue(0, sr_a, dr_a, gs_a)

        def pair(g, carry):
            b0 = 2 * g
            issue(b0 + 1, sr_b, dr_b, gs_b)
            wait_gather(sr_a, dr_a, gs_a)
            compute(b0, sr_a, dr_a, at_a, ws_a, g > 0)
            issue(b0 + 2, sr_a, dr_a, gs_a)
            wait_gather(sr_b, dr_b, gs_b)
            compute(b0 + 1, sr_b, dr_b, at_b, ws_b, g > 0)
            return carry

        lax.fori_loop(0, _PAIRS1, pair, 0)
        wait_gather(sr_a, dr_a, gs_a)
        compute(_NB1 - 1, sr_a, dr_a, at_a, ws_a, _PAIRS1 > 0)
        drain_at(at_a, ws_a)
        drain_at(at_b, ws_b)

    return k(src_tab, dst_tab, esrc, edst, a_scaled)


def _agg_sc(h1cat, esrc, edst, attn):
    mesh = plsc.VectorSubcoreMesh(core_axis_name="c", subcore_axis_name="s")

    @functools.partial(
        pl.kernel,
        out_type=jax.ShapeDtypeStruct((_NC, _N, _HF), jnp.float32),
        mesh=mesh,
        compiler_params=_SC_PARAMS,
        scratch_types=[
            pltpu.VMEM((_EB2, _HF // 2), jnp.int32),
            pltpu.VMEM((_EB2, _HF // 2), jnp.int32),
            pltpu.VMEM((_EB2 * _H,), jnp.float32),
            pltpu.VMEM((_EB2 * _H,), jnp.float32),
            pltpu.VMEM((_EB2, _HF), jnp.float32),
            pltpu.VMEM((_EB2, _HF), jnp.float32),
            pltpu.VMEM((_EB2,), jnp.int32),
            pltpu.VMEM((_EB2,), jnp.int32),
            pltpu.VMEM((_EB2,), jnp.int32),
            pltpu.VMEM((_EB2,), jnp.int32),
            pltpu.VMEM((_EB2,), jnp.int32),
            pltpu.VMEM((_EB2,), jnp.int32),
            pltpu.VMEM((_ZROWS, _HF), jnp.float32),
            pltpu.VMEM_SHARED((_N, _HF), jnp.float32),
            pltpu.SemaphoreType.DMA,
            pltpu.SemaphoreType.DMA,
            pltpu.SemaphoreType.DMA,
            pltpu.SemaphoreType.DMA,
            pltpu.SemaphoreType.DMA,
            pltpu.SemaphoreType.DMA,
        ],
    )
    def k(h1_hbm, esrc_hbm, edst_hbm, attn_hbm, out_hbm,
          rows_a, rows_b, at_a, at_b, msg_a, msg_b, sidx_a, sidx_b,
          didxf_a, didxf_b, didxu_a, didxu_b, zero_v, acc_sh,
          gs_a, gs_b, ss_a, ss_b, is_a, is_b):
        c = lax.axis_index("c")
        s = lax.axis_index("s")
        evens = lax.iota(jnp.int32, 16) * 2
        zvec = jnp.zeros((16,), jnp.float32)

        def zrow(i, carry):
            for kk in range(_HF // 16):
                zero_v[i, pl.ds(kk * 16, 16)] = zvec
            return carry

        lax.fori_loop(0, _ZROWS, zrow, 0)
        for j in range(_ROWS_PT // _ZROWS):
            pltpu.sync_copy(
                zero_v, acc_sh.at[pl.ds(s * _ROWS_PT + j * _ZROWS, _ZROWS)])
        plsc.subcore_barrier()

        ebase = s * _P2_EPT
        cn = c * _N
        hbase = c * (_H // 2)

        def idx_issue(b, sidx, didxf, isem):
            off = ebase + b * _EB2
            pltpu.async_copy(esrc_hbm.at[pl.ds(off, _EB2)], sidx, isem)
            pltpu.async_copy(edst_hbm.at[pl.ds(off, _EB2)], didxf, isem)

        def wait_idx(sidx, didxf, isem):
            pltpu.make_async_copy(
                esrc_hbm.at[pl.ds(0, _EB2)], sidx, isem).wait()
            pltpu.make_async_copy(
                edst_hbm.at[pl.ds(0, _EB2)], didxf, isem).wait()

        def gather_issue(b, sidx, rows, at, gs):
            for kk in range(_EB2 // 16):
                sidx[pl.ds(kk * 16, 16)] = sidx[pl.ds(kk * 16, 16)] + cn
            pltpu.async_copy(h1_hbm.at[sidx], rows, gs)
            pltpu.async_copy(
                attn_hbm.at[pl.ds((ebase + b * _EB2) * _H, _EB2 * _H)], at, gs)

        def wait_gather(rows, at, gs):
            pltpu.make_async_copy(h1_hbm.at[pl.ds(0, _EB2)], rows, gs).wait()
            pltpu.make_async_copy(
                attn_hbm.at[pl.ds(0, _EB2 * _H)], at, gs).wait()

        def wait_scatter(msg, didxu, ss):
            pltpu.make_async_copy(msg, acc_sh.at[didxu], ss).wait()

        def compute(rows, at, msg, didxf, didxu, ss):
            for kk in range(_EB2 // 16):
                didxu[pl.ds(kk * 16, 16)] = didxf[pl.ds(kk * 16, 16)]

            def ebody(i, carry):
                for k4 in range(4):
                    e = i * 4 + k4
                    e8 = e * _H
                    ev = jnp.full((16,), e, jnp.int32)
                    for hh in range(_H // 2):
                        aidx = jnp.full((16,), e8 + hbase + hh, jnp.int32)
                        av = plsc.load_gather(at, [aidx])
                        w = rows[e, pl.ds(hh * 16, 16)]
                        pe, po = plsc.unpack(
                            plsc.bitcast(w, jnp.bfloat16),
                            format=plsc.PackFormat.INTERLEAVED,
                            preferred_element_type=jnp.float32)
                        cole = evens + (hh * _D)
                        plsc.store_scatter(msg, [ev, cole], pe * av)
                        plsc.store_scatter(msg, [ev, cole + 1], po * av)
                return carry

            lax.fori_loop(0, _EB2 // 4, ebody, 0)
            pltpu.async_copy(msg, acc_sh.at[didxu], ss, add=True)

        idx_issue(0, sidx_a, didxf_a, is_a)
        idx_issue(1, sidx_b, didxf_b, is_b)
        wait_idx(sidx_a, didxf_a, is_a)
        gather_issue(0, sidx_a, rows_a, at_a, gs_a)

        def pair(g, carry):
            b0 = 2 * g
            wait_idx(sidx_b, didxf_b, is_b)
            gather_issue(b0 + 1, sidx_b, rows_b, at_b, gs_b)

            @pl.when(g > 0)
            def _():
                wait_scatter(msg_a, didxu_a, ss_a)

            wait_gather(rows_a, at_a, gs_a)
            compute(rows_a, at_a, msg_a, didxf_a, didxu_a, ss_a)
            idx_issue(b0 + 2, sidx_a, didxf_a, is_a)
            wait_idx(sidx_a, didxf_a, is_a)
            gather_issue(b0 + 2, sidx_a, rows_a, at_a, gs_a)

            @pl.when(g > 0)
            def _():
                wait_scatter(msg_b, didxu_b, ss_b)

            wait_gather(rows_b, at_b, gs_b)
            compute(rows_b, at_b, msg_b, didxf_b, didxu_b, ss_b)
            idx_issue(b0 + 3, sidx_b, didxf_b, is_b)
            return carry

        lax.fori_loop(0, _PAIRS2 - 1, pair, 0)
        wait_idx(sidx_b, didxf_b, is_b)
        gather_issue(_NB2 - 1, sidx_b, rows_b, at_b, gs_b)
        wait_scatter(msg_a, didxu_a, ss_a)
        wait_gather(rows_a, at_a, gs_a)
        compute(rows_a, at_a, msg_a, didxf_a, didxu_a, ss_a)
        wait_scatter(msg_b, didxu_b, ss_b)
        wait_gather(rows_b, at_b, gs_b)
        compute(rows_b, at_b, msg_b, didxf_b, didxu_b, ss_b)
        wait_scatter(msg_a, didxu_a, ss_a)
        wait_scatter(msg_b, didxu_b, ss_b)
        plsc.subcore_barrier()
        pltpu.sync_copy(acc_sh.at[pl.ds(s * _ROWS_PT, _ROWS_PT)],
                        out_hbm.at[c, pl.ds(s * _ROWS_PT, _ROWS_PT)])

    return k(h1cat, esrc, edst, attn)


def _pack_bf16(t):
    n, w = t.shape
    return lax.bitcast_convert_type(
        t.astype(jnp.bfloat16).reshape(n, w // 2, 2), jnp.int32)


def _gat_layer(x, edge_index, w1, b1, w2, b2, a):
    esrc = edge_index[0]
    edst = edge_index[1]
    src_tab, dst_tab, h1a, h1b = _tables(x, w1, w2, b1, b2)
    h1cat = jnp.concatenate([h1a, h1b], axis=0)
    a_scaled = (a / math.sqrt(_D)).reshape(-1).astype(jnp.float32)
    attn = _attn_sc(_pack_bf16(src_tab), _pack_bf16(dst_tab),
                    esrc, edst, a_scaled)
    return _agg_sc(_pack_bf16(h1cat), esrc, edst, attn)


def kernel(x, edge_index, W1_0, b1_0, W2_0, b2_0, W3_0, b3_0, a_0, ln_g_0,
           ln_b_0, W1_1, b1_1, W2_1, b2_1, W3_1, b3_1, a_1, ln_g_1, ln_b_1,
           W_out, b_out):
    gat0 = _gat_layer(x, edge_index, W1_0, b1_0, W2_0, b2_0, a_0)
    h = _ln_elu(gat0, ln_g_0, ln_b_0, None)
    gat1 = _gat_layer(h, edge_index, W1_1, b1_1, W2_1, b2_1, a_1)
    h2 = _ln_elu(gat1, ln_g_1, ln_b_1, h)
    return _final(h2, W_out, b_out)


# speedup vs baseline: 1.7213x; 1.0430x over previous
import functools
import math

import jax
import jax.numpy as jnp
from jax import lax
from jax.experimental import pallas as pl
from jax.experimental.pallas import tpu as pltpu
from jax.experimental.pallas import tpu_sc as plsc

_N = 10000
_E = 320000
_H = 8
_D = 32
_HD = _H * _D
_HF = _HD // 2

_NC = 2
_NS = 16
_NW = _NC * _NS

_P1_EPT = _E // _NS
_EB1 = 32
_NB1 = _P1_EPT // _EB1
_PAIRS1 = (_NB1 - 1) // 2

_P2_EPT = _E // _NS
_EB2 = 80
_NB2 = _P2_EPT // _EB2
_PAIRS2 = _NB2 // 2

_ROWS_PT = _N // _NS
_ZROWS = 25

_ROW_BLK = 1000

_SC_PARAMS = pltpu.CompilerParams(
    use_tc_tiling_on_sc=False, needs_layout_passes=False)


def _tables_body(x_ref, w1_ref, w2_ref, b1_ref, b2_ref, h1_ref, h2_ref):
    x = x_ref[...]
    dn = (((1,), (1,)), ((), ()))
    h1_ref[...] = lax.dot_general(x, w1_ref[...], dn,
                                  preferred_element_type=jnp.float32) + b1_ref[...]
    h2_ref[...] = lax.dot_general(x, w2_ref[...], dn,
                                  preferred_element_type=jnp.float32) + b2_ref[...]


def _tables(x, w1, w2, b1, b2):
    n, k = x.shape
    r = _ROW_BLK
    return pl.pallas_call(
        _tables_body,
        grid=(n // r,),
        in_specs=[
            pl.BlockSpec((r, k), lambda i: (i, 0)),
            pl.BlockSpec((_HD, k), lambda i: (0, 0)),
            pl.BlockSpec((_HD, k), lambda i: (0, 0)),
            pl.BlockSpec((1, _HD), lambda i: (0, 0)),
            pl.BlockSpec((1, _HD), lambda i: (0, 0)),
        ],
        out_specs=[
            pl.BlockSpec((r, _HD), lambda i: (i, 0)),
            pl.BlockSpec((r, _HD), lambda i: (i, 0)),
        ],
        out_shape=[
            jax.ShapeDtypeStruct((n, _HD), jnp.float32),
            jax.ShapeDtypeStruct((n, _HD), jnp.float32),
        ],
    )(x, w1, w2, b1.reshape(1, -1), b2.reshape(1, -1))


def _ln_elu_body(has_res, ha_ref, hb_ref, g_ref, be_ref, *rest):
    if has_res:
        res_ref, o_ref = rest
    else:
        (o_ref,) = rest
    h = jnp.concatenate([ha_ref[0], hb_ref[0]], axis=1)
    m = jnp.mean(h, axis=1, keepdims=True)
    xm = h - m
    v = jnp.mean(xm * xm, axis=1, keepdims=True)
    y = xm * lax.rsqrt(v + 1e-5) * g_ref[...] + be_ref[...]
    y = jnp.where(y > 0, y, jnp.exp(y) - 1.0)
    if has_res:
        y = y + res_ref[...]
    o_ref[...] = y


def _ln_elu(gat2, g, b, res):
    r = _ROW_BLK
    has_res = res is not None
    in_specs = [
        pl.BlockSpec((1, r, _HF), lambda i: (0, i, 0)),
        pl.BlockSpec((1, r, _HF), lambda i: (1, i, 0)),
        pl.BlockSpec((1, _HD), lambda i: (0, 0)),
        pl.BlockSpec((1, _HD), lambda i: (0, 0)),
    ]
    args = [gat2, gat2, g.reshape(1, -1), b.reshape(1, -1)]
    if has_res:
        in_specs.append(pl.BlockSpec((r, _HD), lambda i: (i, 0)))
        args.append(res)
    return pl.pallas_call(
        functools.partial(_ln_elu_body, has_res),
        grid=(_N // r,),
        in_specs=in_specs,
        out_specs=pl.BlockSpec((r, _HD), lambda i: (i, 0)),
        out_shape=jax.ShapeDtypeStruct((_N, _HD), jnp.float32),
    )(*args)


def _final_body(h_ref, w_ref, b_ref, o_ref):
    dn = (((1,), (1,)), ((), ()))
    o_ref[...] = lax.dot_general(h_ref[...], w_ref[...], dn,
                                 preferred_element_type=jnp.float32) + b_ref[...]


def _final(h, w_out, b_out):
    r = _ROW_BLK
    d_out = w_out.shape[0]
    return pl.pallas_call(
        _final_body,
        grid=(_N // r,),
        in_specs=[
            pl.BlockSpec((r, _HD), lambda i: (i, 0)),
            pl.BlockSpec((d_out, _HD), lambda i: (0, 0)),
            pl.BlockSpec((1, d_out), lambda i: (0, 0)),
        ],
        out_specs=pl.BlockSpec((r, d_out), lambda i: (i, 0)),
        out_shape=jax.ShapeDtypeStruct((_N, d_out), jnp.float32),
    )(h, w_out, b_out.reshape(1, -1))


def _attn_sc(tab, esrc, edst, a_scaled):
    mesh = plsc.VectorSubcoreMesh(core_axis_name="c", subcore_axis_name="s")

    @functools.partial(
        pl.kernel,
        out_type=jax.ShapeDtypeStruct((_NC * _E * (_H // 2),), jnp.float32),
        mesh=mesh,
        compiler_params=_SC_PARAMS,
        scratch_types=[
            pltpu.VMEM((_EB1,), jnp.int32),
            pltpu.VMEM((_EB1,), jnp.int32),
            pltpu.VMEM((_EB1,), jnp.int32),
            pltpu.VMEM((_EB1,), jnp.int32),
            pltpu.VMEM((_EB1, _HF), jnp.int32),
            pltpu.VMEM((_EB1, _HF), jnp.int32),
            pltpu.VMEM((_EB1, _HF), jnp.int32),
            pltpu.VMEM((_EB1, _HF), jnp.int32),
            pltpu.VMEM((_EB1 * (_H // 2),), jnp.float32),
            pltpu.VMEM((_EB1 * (_H // 2),), jnp.float32),
            pltpu.VMEM((_HD,), jnp.float32),
            pltpu.VMEM_SHARED((_N, _HF), jnp.int32),
            pltpu.SemaphoreType.DMA,
            pltpu.SemaphoreType.DMA,
            pltpu.SemaphoreType.DMA,
            pltpu.SemaphoreType.DMA,
            pltpu.SemaphoreType.DMA,
            pltpu.SemaphoreType.DMA,
        ],
    )
    def k(tab_hbm, esrc_hbm, edst_hbm, a_hbm, lg_hbm,
          sidx_a, sidx_b, didx_a, didx_b, sr_a, sr_b, dr_a, dr_b,
          lt_a, lt_b, a_v, tab_sh, is_a, is_b, gs_a, gs_b, ws_a, ws_b):
        c = lax.axis_index("c")
        s = lax.axis_index("s")
        pltpu.sync_copy(tab_hbm.at[pl.ds(c * _N + s * _ROWS_PT, _ROWS_PT)],
                        tab_sh.at[pl.ds(s * _ROWS_PT, _ROWS_PT)])
        pltpu.sync_copy(a_hbm, a_v)
        plsc.subcore_barrier()
        lanes = lax.iota(jnp.int32, 16)
        lanes4 = lanes * 4
        zero16 = jnp.zeros((16,), jnp.float32)
        ebase = s * _P1_EPT
        lbase = c * (_E * (_H // 2)) + ebase * (_H // 2)
        c128 = c * _HF

        def idx_issue(b, sidx, didx, isem):
            off = ebase + b * _EB1
            pltpu.async_copy(esrc_hbm.at[pl.ds(off, _EB1)], sidx, isem)
            pltpu.async_copy(edst_hbm.at[pl.ds(off, _EB1)], didx, isem)

        def wait_idx(sidx, didx, isem):
            pltpu.make_async_copy(
                esrc_hbm.at[pl.ds(0, _EB1)], sidx, isem).wait()
            pltpu.make_async_copy(
                edst_hbm.at[pl.ds(0, _EB1)], didx, isem).wait()

        def gather_issue(sidx, didx, sr, dr, gs):
            pltpu.async_copy(tab_sh.at[sidx], sr, gs)
            pltpu.async_copy(tab_sh.at[didx], dr, gs)

        def wait_gather(sr, dr, gs):
            pltpu.make_async_copy(tab_hbm.at[pl.ds(0, _EB1)], sr, gs).wait()
            pltpu.make_async_copy(tab_hbm.at[pl.ds(0, _EB1)], dr, gs).wait()

        def drain_lt(lt, ws):
            pltpu.make_async_copy(
                lt, lg_hbm.at[pl.ds(0, _EB1 * (_H // 2))], ws).wait()

        def unp(w):
            return plsc.unpack(plsc.bitcast(w, jnp.bfloat16),
                               format=plsc.PackFormat.INTERLEAVED,
                               preferred_element_type=jnp.float32)

        def compute(b, sr, dr, lt, ws):
            for sb in range(_EB1 // 16):
                rid = lanes + sb * 16
                accs = (zero16,) * (_H // 2)

                def dbody(d2, accs):
                    out = []
                    for hh in range(_H // 2):
                        colv = jnp.full((16,), d2 + hh * (_D // 2), jnp.int32)
                        colv2 = colv + _HF // 2
                        h1se, h1so = unp(plsc.load_gather(sr, [rid, colv]))
                        h2se, h2so = unp(plsc.load_gather(sr, [rid, colv2]))
                        h1de, h1do = unp(plsc.load_gather(dr, [rid, colv]))
                        h2de, h2do = unp(plsc.load_gather(dr, [rid, colv2]))
                        ze = h1se + h2de + (h1se * h2se) * (h1de * h2de)
                        zo = h1so + h2do + (h1so * h2so) * (h1do * h2do)
                        eze = jnp.where(ze > 0, ze, jnp.exp(ze) - 1.0)
                        ezo = jnp.where(zo > 0, zo, jnp.exp(zo) - 1.0)
                        colva = c128 + 2 * d2 + hh * _D
                        cave = jnp.full((16,), colva, jnp.int32)
                        ave = plsc.load_gather(a_v, [cave])
                        avo = plsc.load_gather(a_v, [cave + 1])
                        out.append(accs[hh] + (ave * eze + avo * ezo))
                    return tuple(out)

                accs = lax.fori_loop(0, _D // 2, dbody, accs)
                for hh in range(_H // 2):
                    plsc.store_scatter(
                        lt, [lanes4 + (sb * 64 + hh)], accs[hh])
            pltpu.async_copy(
                lt,
                lg_hbm.at[pl.ds(lbase + b * (_EB1 * (_H // 2)),
                                _EB1 * (_H // 2))],
                ws)

        idx_issue(0, sidx_a, didx_a, is_a)
        idx_issue(1, sidx_b, didx_b, is_b)
        wait_idx(sidx_a, didx_a, is_a)
        gather_issue(sidx_a, didx_a, sr_a, dr_a, gs_a)

        def pair(g, carry):
            b0 = 2 * g
            wait_idx(sidx_b, didx_b, is_b)
            gather_issue(sidx_b, didx_b, sr_b, dr_b, gs_b)

            @pl.when(g > 0)
            def _():
                drain_lt(lt_a, ws_a)

            wait_gather(sr_a, dr_a, gs_a)
            compute(b0, sr_a, dr_a, lt_a, ws_a)
            idx_issue(b0 + 2, sidx_a, didx_a, is_a)

            @pl.when(g > 0)
            def _():
                drain_lt(lt_b, ws_b)

            wait_gather(sr_b, dr_b, gs_b)
            compute(b0 + 1, sr_b, dr_b, lt_b, ws_b)
            wait_idx(sidx_a, didx_a, is_a)
            gather_issue(sidx_a, didx_a, sr_a, dr_a, gs_a)
            idx_issue(jnp.minimum(b0 + 3, _NB1 - 1), sidx_b, didx_b, is_b)
            return carry

        lax.fori_loop(0, _PAIRS1, pair, 0)
        wait_idx(sidx_b, didx_b, is_b)
        drain_lt(lt_a, ws_a)
        wait_gather(sr_a, dr_a, gs_a)
        compute(_NB1 - 1, sr_a, dr_a, lt_a, ws_a)
        drain_lt(lt_a, ws_a)
        drain_lt(lt_b, ws_b)

    return k(tab, esrc, edst, a_scaled)


def _agg_sc(h1cat, esrc, edst, lg):
    mesh = plsc.VectorSubcoreMesh(core_axis_name="c", subcore_axis_name="s")

    @functools.partial(
        pl.kernel,
        out_type=jax.ShapeDtypeStruct((_NC, _N, _HF), jnp.float32),
        mesh=mesh,
        compiler_params=_SC_PARAMS,
        scratch_types=[
            pltpu.VMEM((_EB2, _HF // 2), jnp.int32),
            pltpu.VMEM((_EB2, _HF // 2), jnp.int32),
            pltpu.VMEM((_EB2 * (_H // 2),), jnp.float32),
            pltpu.VMEM((_EB2 * (_H // 2),), jnp.float32),
            pltpu.VMEM((_EB2 * (_H // 2),), jnp.float32),
            pltpu.VMEM((_EB2 * (_H // 2),), jnp.float32),
            pltpu.VMEM((_EB2 * _H,), jnp.float32),
            pltpu.VMEM((_EB2 * _H,), jnp.float32),
            pltpu.VMEM((_EB2, _HF), jnp.float32),
            pltpu.VMEM((_EB2, _HF), jnp.float32),
            pltpu.VMEM((_EB2,), jnp.int32),
            pltpu.VMEM((_EB2,), jnp.int32),
            pltpu.VMEM((_EB2,), jnp.int32),
            pltpu.VMEM((_EB2,), jnp.int32),
            pltpu.VMEM((_EB2,), jnp.int32),
            pltpu.VMEM((_EB2,), jnp.int32),
            pltpu.VMEM((_ZROWS, _HF), jnp.float32),
            pltpu.VMEM_SHARED((_N, _HF), jnp.float32),
            pltpu.SemaphoreType.DMA,
            pltpu.SemaphoreType.DMA,
            pltpu.SemaphoreType.DMA,
            pltpu.SemaphoreType.DMA,
            pltpu.SemaphoreType.DMA,
            pltpu.SemaphoreType.DMA,
        ],
    )
    def k(h1_hbm, esrc_hbm, edst_hbm, lg_hbm, out_hbm,
          rows_a, rows_b, lg0_a, lg0_b, lg1_a, lg1_b, at_a, at_b,
          msg_a, msg_b, sidx_a, sidx_b,
          didxf_a, didxf_b, didxu_a, didxu_b, zero_v, acc_sh,
          gs_a, gs_b, ss_a, ss_b, is_a, is_b):
        c = lax.axis_index("c")
        s = lax.axis_index("s")
        lanes = lax.iota(jnp.int32, 16)
        evens = lanes * 2
        lanes4 = lanes * 4
        lanes8 = lanes * _H
        zvec = jnp.zeros((16,), jnp.float32)

        def zrow(i, carry):
            for kk in range(_HF // 16):
                zero_v[i, pl.ds(kk * 16, 16)] = zvec
            return carry

        lax.fori_loop(0, _ZROWS, zrow, 0)
        for j in range(_ROWS_PT // _ZROWS):
            pltpu.sync_copy(
                zero_v, acc_sh.at[pl.ds(s * _ROWS_PT + j * _ZROWS, _ZROWS)])
        plsc.subcore_barrier()

        ebase = s * _P2_EPT
        cn = c * _N
        hbase = c * (_H // 2)

        def idx_issue(b, sidx, didxf, isem):
            off = ebase + b * _EB2
            pltpu.async_copy(esrc_hbm.at[pl.ds(off, _EB2)], sidx, isem)
            pltpu.async_copy(edst_hbm.at[pl.ds(off, _EB2)], didxf, isem)

        def wait_idx(sidx, didxf, isem):
            pltpu.make_async_copy(
                esrc_hbm.at[pl.ds(0, _EB2)], sidx, isem).wait()
            pltpu.make_async_copy(
                edst_hbm.at[pl.ds(0, _EB2)], didxf, isem).wait()

        def gather_issue(b, sidx, rows, lg0, lg1, gs):
            for kk in range(_EB2 // 16):
                sidx[pl.ds(kk * 16, 16)] = sidx[pl.ds(kk * 16, 16)] + cn
            pltpu.async_copy(h1_hbm.at[sidx], rows, gs)
            off4 = (ebase + b * _EB2) * (_H // 2)
            nl = _EB2 * (_H // 2)
            pltpu.async_copy(lg_hbm.at[pl.ds(off4, nl)], lg0, gs)
            pltpu.async_copy(
                lg_hbm.at[pl.ds(_E * (_H // 2) + off4, nl)], lg1, gs)

        def wait_gather(rows, lg0, lg1, gs):
            nl = _EB2 * (_H // 2)
            pltpu.make_async_copy(h1_hbm.at[pl.ds(0, _EB2)], rows, gs).wait()
            pltpu.make_async_copy(lg_hbm.at[pl.ds(0, nl)], lg0, gs).wait()
            pltpu.make_async_copy(lg_hbm.at[pl.ds(0, nl)], lg1, gs).wait()

        def wait_scatter(msg, didxu, ss):
            pltpu.make_async_copy(msg, acc_sh.at[didxu], ss).wait()

        def compute(rows, lg0, lg1, at, msg, didxf, didxu, ss):
            for kk in range(_EB2 // 16):
                didxu[pl.ds(kk * 16, 16)] = didxf[pl.ds(kk * 16, 16)]
            for sb in range(_EB2 // 16):
                ls = []
                for j in range(_H):
                    src = lg0 if j < _H // 2 else lg1
                    idx = lanes4 + (sb * 64 + (j % (_H // 2)))
                    ls.append(plsc.load_gather(src, [idx]))
                m = ls[0]
                for j in range(1, _H):
                    m = jnp.maximum(m, ls[j])
                es = [jnp.exp(v - m) for v in ls]
                tot = es[0]
                for j in range(1, _H):
                    tot = tot + es[j]
                r = 1.0 / tot
                for j in range(_H):
                    plsc.store_scatter(
                        at, [lanes8 + (sb * 128 + j)], es[j] * r)

            def ebody(i, carry):
                for k4 in range(4):
                    e = i * 4 + k4
                    e8 = e * _H
                    ev = jnp.full((16,), e, jnp.int32)
                    for hh in range(_H // 2):
                        aidx = jnp.full((16,), e8 + hbase + hh, jnp.int32)
                        av = plsc.load_gather(at, [aidx])
                        w = rows[e, pl.ds(hh * 16, 16)]
                        pe, po = plsc.unpack(
                            plsc.bitcast(w, jnp.bfloat16),
                            format=plsc.PackFormat.INTERLEAVED,
                            preferred_element_type=jnp.float32)
                        cole = evens + (hh * _D)
                        plsc.store_scatter(msg, [ev, cole], pe * av)
                        plsc.store_scatter(msg, [ev, cole + 1], po * av)
                return carry

            lax.fori_loop(0, _EB2 // 4, ebody, 0)
            pltpu.async_copy(msg, acc_sh.at[didxu], ss, add=True)

        idx_issue(0, sidx_a, didxf_a, is_a)
        idx_issue(1, sidx_b, didxf_b, is_b)
        wait_idx(sidx_a, didxf_a, is_a)
        gather_issue(0, sidx_a, rows_a, lg0_a, lg1_a, gs_a)

        def pair(g, carry):
            b0 = 2 * g
            wait_idx(sidx_b, didxf_b, is_b)
            gather_issue(b0 + 1, sidx_b, rows_b, lg0_b, lg1_b, gs_b)

            @pl.when(g > 0)
            def _():
                wait_scatter(msg_a, didxu_a, ss_a)

            wait_gather(rows_a, lg0_a, lg1_a, gs_a)
            compute(rows_a, lg0_a, lg1_a, at_a, msg_a, didxf_a, didxu_a, ss_a)
            idx_issue(b0 + 2, sidx_a, didxf_a, is_a)
            wait_idx(sidx_a, didxf_a, is_a)
            gather_issue(b0 + 2, sidx_a, rows_a, lg0_a, lg1_a, gs_a)

            @pl.when(g > 0)
            def _():
                wait_scatter(msg_b, didxu_b, ss_b)

            wait_gather(rows_b, lg0_b, lg1_b, gs_b)
            compute(rows_b, lg0_b, lg1_b, at_b, msg_b, didxf_b, didxu_b, ss_b)
            idx_issue(b0 + 3, sidx_b, didxf_b, is_b)
            return carry

        lax.fori_loop(0, _PAIRS2 - 1, pair, 0)
        wait_idx(sidx_b, didxf_b, is_b)
        gather_issue(_NB2 - 1, sidx_b, rows_b, lg0_b, lg1_b, gs_b)
        wait_scatter(msg_a, didxu_a, ss_a)
        wait_gather(rows_a, lg0_a, lg1_a, gs_a)
        compute(rows_a, lg0_a, lg1_a, at_a, msg_a, didxf_a, didxu_a, ss_a)
        wait_scatter(msg_b, didxu_b, ss_b)
        wait_gather(rows_b, lg0_b, lg1_b, gs_b)
        compute(rows_b, lg0_b, lg1_b, at_b, msg_b, didxf_b, didxu_b, ss_b)
        wait_scatter(msg_a, didxu_a, ss_a)
        wait_scatter(msg_b, didxu_b, ss_b)
        plsc.subcore_barrier()
        pltpu.sync_copy(acc_sh.at[pl.ds(s * _ROWS_PT, _ROWS_PT)],
                        out_hbm.at[c, pl.ds(s * _ROWS_PT, _ROWS_PT)])

    return k(h1cat, esrc, edst, lg)


def _pack_bf16(t):
    n, w = t.shape
    return lax.bitcast_convert_type(
        t.astype(jnp.bfloat16).reshape(n, w // 2, 2), jnp.int32)


def _gat_layer(x, edge_index, w1, b1, w2, b2, a):
    esrc = edge_index[0]
    edst = edge_index[1]
    h1, h2 = _tables(x, w1, w2, b1, b2)
    tab = jnp.concatenate(
        [jnp.concatenate([h1[:, :_HF], h2[:, :_HF]], axis=1),
         jnp.concatenate([h1[:, _HF:], h2[:, _HF:]], axis=1)], axis=0)
    h1cat = jnp.concatenate([h1[:, :_HF], h1[:, _HF:]], axis=0)
    a_scaled = (a / math.sqrt(_D)).reshape(-1).astype(jnp.float32)
    lg = _attn_sc(_pack_bf16(tab), esrc, edst, a_scaled)
    return _agg_sc(_pack_bf16(h1cat), esrc, edst, lg)


def kernel(x, edge_index, W1_0, b1_0, W2_0, b2_0, W3_0, b3_0, a_0, ln_g_0,
           ln_b_0, W1_1, b1_1, W2_1, b2_1, W3_1, b3_1, a_1, ln_g_1, ln_b_1,
           W_out, b_out):
    gat0 = _gat_layer(x, edge_index, W1_0, b1_0, W2_0, b2_0, a_0)
    h = _ln_elu(gat0, ln_g_0, ln_b_0, None)
    gat1 = _gat_layer(h, edge_index, W1_1, b1_1, W2_1, b2_1, a_1)
    h2 = _ln_elu(gat1, ln_g_1, ln_b_1, h)
    return _final(h2, W_out, b_out)
